# baseline (speedup 1.0000x reference)
import jax
import jax.numpy as jnp
from jax.experimental import pallas as pl
from jax.experimental.pallas import tpu as pltpu

_TM = 256


def _linear_kernel(x_ref, w_ref, b_ref, o_ref, wb_ref):
    i = pl.program_id(0)

    @pl.when(i == 0)
    def _():
        wb_ref[...] = w_ref[...].astype(jnp.bfloat16)

    o_ref[...] = (
        jax.lax.dot_general(
            x_ref[...], wb_ref[...],
            dimension_numbers=(((1,), (1,)), ((), ())),
            preferred_element_type=jnp.float32,
        )
        + b_ref[...]
    )


@jax.jit
def _forward(x, weight, bias):
    B = x.shape[0]
    F = x.shape[1] * x.shape[2] * x.shape[3]
    O = weight.shape[0]

    x_flat = x.reshape(B, F).astype(jnp.bfloat16)
    b2 = bias.reshape(1, O)
    grid_m = B // _TM

    return pl.pallas_call(
        _linear_kernel,
        out_shape=jax.ShapeDtypeStruct((B, O), jnp.float32),
        grid=(grid_m,),
        in_specs=[
            pl.BlockSpec((_TM, F), lambda i: (i, 0)),
            pl.BlockSpec((O, F), lambda i: (0, 0)),
            pl.BlockSpec((1, O), lambda i: (0, 0)),
        ],
        out_specs=pl.BlockSpec((_TM, O), lambda i: (i, 0)),
        scratch_shapes=[
            pltpu.VMEM((O, F), jnp.bfloat16),
        ],
        compiler_params=pltpu.CompilerParams(
            dimension_semantics=("arbitrary",),
            vmem_limit_bytes=40 << 20,
        ),
    )(x_flat, weight, b2)


def kernel(x, weight, bias):
    return _forward(x, weight, bias)


# speedup vs baseline: 1.1445x; 1.1445x over previous
import jax
import jax.numpy as jnp
from jax.experimental import pallas as pl
from jax.experimental.pallas import tpu as pltpu

_TM = 512


def _linear_kernel(x_ref, w_ref, b_ref, o_ref):
    o_ref[...] = (
        jax.lax.dot_general(
            x_ref[...], w_ref[...],
            dimension_numbers=(((1,), (1,)), ((), ())),
            preferred_element_type=jnp.float32,
        )
        + b_ref[...]
    )


@jax.jit
def _forward(x, weight, bias):
    B = x.shape[0]
    F = x.shape[1] * x.shape[2] * x.shape[3]
    O = weight.shape[0]

    x_flat = x.reshape(B, F)
    b2 = bias.reshape(1, O)
    grid_m = B // _TM

    return pl.pallas_call(
        _linear_kernel,
        out_shape=jax.ShapeDtypeStruct((B, O), jnp.float32),
        grid=(grid_m,),
        in_specs=[
            pl.BlockSpec((_TM, F), lambda i: (i, 0)),
            pl.BlockSpec((O, F), lambda i: (0, 0)),
            pl.BlockSpec((1, O), lambda i: (0, 0)),
        ],
        out_specs=pl.BlockSpec((_TM, O), lambda i: (i, 0)),
        compiler_params=pltpu.CompilerParams(
            dimension_semantics=("parallel",),
            vmem_limit_bytes=40 << 20,
        ),
    )(x_flat, weight, b2)


def kernel(x, weight, bias):
    return _forward(x, weight, bias)
